# baseline (speedup 1.0000x reference)
import functools

import jax
import jax.numpy as jnp
from jax import lax
from jax.experimental import pallas as pl
from jax.experimental.pallas import tpu as pltpu
from jax.experimental.pallas import tpu_sc as plsc

_BATCH = 1024
_HIDDEN = 64
_VOCAB = 100000
_VT = 2048

_NC = 2
_NS = 16
_NW = _NC * _NS
_BPW = _BATCH // _NW


def _gather_sc(table, ids):
    mesh = plsc.ScalarSubcoreMesh(axis_name="c", num_cores=_NC)
    per_core = _BATCH // _NC
    chunk = 16

    @functools.partial(
        pl.kernel,
        mesh=mesh,
        out_type=jax.ShapeDtypeStruct((_BATCH, _HIDDEN), jnp.float32),
        scratch_types=[
            pltpu.SMEM((per_core,), jnp.int32),
            pltpu.SemaphoreType.DMA,
            pltpu.SemaphoreType.DMA,
        ],
    )
    def k(table_hbm, idx_hbm, out_hbm, idx_s, sem_in, sem_out):
        cid = lax.axis_index("c")
        base = cid * per_core
        pltpu.async_copy(idx_hbm.at[pl.ds(base, per_core)], idx_s, sem_in).wait()

        @pl.loop(0, per_core, step=chunk)
        def _(c):
            for j in range(chunk):
                rid = idx_s[c + j]
                pltpu.async_copy(
                    table_hbm.at[pl.ds(rid, 1)],
                    out_hbm.at[pl.ds(base + c + j, 1)],
                    sem_out,
                )

        @pl.loop(0, per_core, step=chunk)
        def _(c):
            for j in range(chunk):
                pltpu.make_async_copy(
                    table_hbm.at[pl.ds(0, 1)],
                    out_hbm.at[pl.ds(base + c + j, 1)],
                    sem_out,
                ).wait()

    return k(table, ids)


def _project_body(wT_ref, emb_ref, b_ref, out_ref):
    acc = lax.dot_general(
        wT_ref[...].astype(jnp.bfloat16),
        emb_ref[...].astype(jnp.bfloat16),
        (((0,), (1,)), ((), ())),
        preferred_element_type=jnp.float32,
    )
    out_ref[...] = acc + jnp.transpose(b_ref[...])


def _project(wT, emb, bias2d, interpret=False):
    return pl.pallas_call(
        _project_body,
        grid=(pl.cdiv(_VOCAB, _VT),),
        in_specs=[
            pl.BlockSpec((_HIDDEN, _VT), lambda i: (0, i)),
            pl.BlockSpec((_BATCH, _HIDDEN), lambda i: (0, 0)),
            pl.BlockSpec((1, _VT), lambda i: (0, i)),
        ],
        out_specs=pl.BlockSpec((_VT, _BATCH), lambda i: (i, 0)),
        out_shape=jax.ShapeDtypeStruct((_VOCAB, _BATCH), jnp.float32),
        interpret=interpret,
    )(wT, emb, bias2d)


def kernel(input_ids, embedding_weight, linear_weight, linear_bias):
    ids = input_ids.astype(jnp.int32)
    emb = _gather_sc(embedding_weight, ids)
    bias2d = linear_bias.reshape(1, _VOCAB)
    logitsT = _project(linear_weight.T, emb, bias2d)
    return logitsT.T


# speedup vs baseline: 1.0090x; 1.0090x over previous
import functools

import jax
import jax.numpy as jnp
from jax import lax
from jax.experimental import pallas as pl
from jax.experimental.pallas import tpu as pltpu
from jax.experimental.pallas import tpu_sc as plsc

_BATCH = 1024
_HIDDEN = 64
_VOCAB = 100000
_VT = 4096

_NC = 2
_NS = 16
_NW = _NC * _NS
_BPW = _BATCH // _NW


def _gather_sc(table, ids):
    mesh = plsc.ScalarSubcoreMesh(axis_name="c", num_cores=_NC)
    per_core = _BATCH // _NC
    chunk = 16

    @functools.partial(
        pl.kernel,
        mesh=mesh,
        out_type=jax.ShapeDtypeStruct((_BATCH, _HIDDEN), jnp.float32),
        scratch_types=[
            pltpu.SMEM((per_core,), jnp.int32),
            pltpu.SemaphoreType.DMA,
            pltpu.SemaphoreType.DMA,
        ],
    )
    def k(table_hbm, idx_hbm, out_hbm, idx_s, sem_in, sem_out):
        cid = lax.axis_index("c")
        base = cid * per_core
        pltpu.async_copy(idx_hbm.at[pl.ds(base, per_core)], idx_s, sem_in).wait()

        @pl.loop(0, per_core, step=chunk)
        def _(c):
            for j in range(chunk):
                rid = idx_s[c + j]
                pltpu.async_copy(
                    table_hbm.at[pl.ds(rid, 1)],
                    out_hbm.at[pl.ds(base + c + j, 1)],
                    sem_out,
                )

        @pl.loop(0, per_core, step=chunk)
        def _(c):
            for j in range(chunk):
                pltpu.make_async_copy(
                    table_hbm.at[pl.ds(0, 1)],
                    out_hbm.at[pl.ds(base + c + j, 1)],
                    sem_out,
                ).wait()

    return k(table, ids)


def _project_body(wT_ref, emb_ref, b_ref, out_ref):
    acc = lax.dot_general(
        wT_ref[...].astype(jnp.bfloat16),
        emb_ref[...].astype(jnp.bfloat16),
        (((0,), (1,)), ((), ())),
        preferred_element_type=jnp.float32,
    )
    out_ref[...] = acc + jnp.transpose(b_ref[...])


def _project(wT, emb, bias2d, interpret=False):
    return pl.pallas_call(
        _project_body,
        grid=(pl.cdiv(_VOCAB, _VT),),
        in_specs=[
            pl.BlockSpec((_HIDDEN, _VT), lambda i: (0, i)),
            pl.BlockSpec((_BATCH, _HIDDEN), lambda i: (0, 0)),
            pl.BlockSpec((1, _VT), lambda i: (0, i)),
        ],
        out_specs=pl.BlockSpec((_VT, _BATCH), lambda i: (i, 0)),
        out_shape=jax.ShapeDtypeStruct((_VOCAB, _BATCH), jnp.float32),
        interpret=interpret,
    )(wT, emb, bias2d)


def kernel(input_ids, embedding_weight, linear_weight, linear_bias):
    ids = input_ids.astype(jnp.int32)
    emb = _gather_sc(embedding_weight, ids)
    bias2d = linear_bias.reshape(1, _VOCAB)
    logitsT = _project(linear_weight.T, emb, bias2d)
    return logitsT.T


# speedup vs baseline: 1.0244x; 1.0153x over previous
import functools

import jax
import jax.numpy as jnp
from jax import lax
from jax.experimental import pallas as pl
from jax.experimental.pallas import tpu as pltpu
from jax.experimental.pallas import tpu_sc as plsc

_BATCH = 1024
_HIDDEN = 64
_VOCAB = 100000
_VT = 4096
_PACK_BLK = 1024
_SPLIT = 50176

_NC = 2
_NS = 16
_NW = _NC * _NS
_BPW = _BATCH // _NW


def _pack_body(l_ref, r_ref, out_ref):
    out_ref[...] = jnp.concatenate(
        [jnp.transpose(l_ref[...]), jnp.transpose(r_ref[...])], axis=1
    )


def _pack(tabT, interpret=False):
    nblk = _SPLIT // _PACK_BLK
    return pl.pallas_call(
        _pack_body,
        grid=(nblk,),
        in_specs=[
            pl.BlockSpec((_HIDDEN, _PACK_BLK), lambda i: (0, i)),
            pl.BlockSpec((_HIDDEN, _PACK_BLK), lambda i: (0, i + nblk)),
        ],
        out_specs=pl.BlockSpec((_PACK_BLK, 128), lambda i: (i, 0)),
        out_shape=jax.ShapeDtypeStruct((_SPLIT, 128), jnp.float32),
        interpret=interpret,
    )(tabT, tabT)


def _gather_sc(packed, ids):
    mesh = plsc.VectorSubcoreMesh(core_axis_name="c", subcore_axis_name="s")

    @functools.partial(
        pl.kernel,
        mesh=mesh,
        out_type=jax.ShapeDtypeStruct((_BATCH, 128), jnp.float32),
        scratch_types=[
            pltpu.VMEM((_BPW,), jnp.int32),
            pltpu.VMEM((_BPW, 128), jnp.float32),
            pltpu.SemaphoreType.DMA,
        ],
    )
    def k(tab_hbm, idx_hbm, out_hbm, idx_v, rows_v, sem):
        wid = lax.axis_index("s") * _NC + lax.axis_index("c")
        base = wid * _BPW
        pltpu.sync_copy(idx_hbm.at[pl.ds(base, _BPW)], idx_v)
        pltpu.async_copy(tab_hbm.at[idx_v], rows_v, sem).wait()
        pltpu.sync_copy(rows_v, out_hbm.at[pl.ds(base, _BPW)])

    return k(packed, ids)


def _project_body(wT_ref, emb_ref, hsel_ref, b_ref, out_ref):
    e = emb_ref[...]
    emb = jnp.where(hsel_ref[...] > 0.5, e[:, _HIDDEN:], e[:, :_HIDDEN])
    acc = lax.dot_general(
        wT_ref[...].astype(jnp.bfloat16),
        emb.astype(jnp.bfloat16),
        (((0,), (1,)), ((), ())),
        preferred_element_type=jnp.float32,
    )
    out_ref[...] = acc + jnp.transpose(b_ref[...])


def _project(wT, emb, hsel, bias2d, interpret=False):
    return pl.pallas_call(
        _project_body,
        grid=(pl.cdiv(_VOCAB, _VT),),
        in_specs=[
            pl.BlockSpec((_HIDDEN, _VT), lambda i: (0, i)),
            pl.BlockSpec((_BATCH, 128), lambda i: (0, 0)),
            pl.BlockSpec((_BATCH, 1), lambda i: (0, 0)),
            pl.BlockSpec((1, _VT), lambda i: (0, i)),
        ],
        out_specs=pl.BlockSpec((_VT, _BATCH), lambda i: (i, 0)),
        out_shape=jax.ShapeDtypeStruct((_VOCAB, _BATCH), jnp.float32),
        interpret=interpret,
    )(wT, emb, hsel, bias2d)


def kernel(input_ids, embedding_weight, linear_weight, linear_bias):
    ids = input_ids.astype(jnp.int32)
    packed = _pack(embedding_weight.T)
    rid = jnp.where(ids < _SPLIT, ids, ids - _SPLIT)
    hsel = (ids >= _SPLIT).astype(jnp.float32).reshape(_BATCH, 1)
    emb128 = _gather_sc(packed, rid)
    bias2d = linear_bias.reshape(1, _VOCAB)
    logitsT = _project(linear_weight.T, emb128, hsel, bias2d)
    return logitsT.T


# speedup vs baseline: 1.0375x; 1.0128x over previous
import functools

import jax
import jax.numpy as jnp
from jax import lax
from jax.experimental import pallas as pl
from jax.experimental.pallas import tpu as pltpu
from jax.experimental.pallas import tpu_sc as plsc

_BATCH = 1024
_HIDDEN = 64
_VOCAB = 100000
_VT = 4096
_PACK_BLK = 1024
_SPLIT = 50176

_NC = 2
_NS = 16
_NW = _NC * _NS
_BPW = _BATCH // _NW


def _pack_body(l_ref, r_ref, out_ref):
    eye = jnp.eye(_HIDDEN, dtype=jnp.bfloat16)
    lT = lax.dot_general(
        l_ref[...].astype(jnp.bfloat16), eye,
        (((0,), (0,)), ((), ())), preferred_element_type=jnp.float32,
    )
    rT = lax.dot_general(
        r_ref[...].astype(jnp.bfloat16), eye,
        (((0,), (0,)), ((), ())), preferred_element_type=jnp.float32,
    )
    out_ref[...] = jnp.concatenate([lT, rT], axis=1)


def _pack(tabT, interpret=False):
    nblk = _SPLIT // _PACK_BLK
    return pl.pallas_call(
        _pack_body,
        grid=(nblk,),
        in_specs=[
            pl.BlockSpec((_HIDDEN, _PACK_BLK), lambda i: (0, i)),
            pl.BlockSpec((_HIDDEN, _PACK_BLK), lambda i: (0, i + nblk)),
        ],
        out_specs=pl.BlockSpec((_PACK_BLK, 128), lambda i: (i, 0)),
        out_shape=jax.ShapeDtypeStruct((_SPLIT, 128), jnp.float32),
        interpret=interpret,
    )(tabT, tabT)


def _gather_sc(packed, ids):
    mesh = plsc.VectorSubcoreMesh(core_axis_name="c", subcore_axis_name="s")

    @functools.partial(
        pl.kernel,
        mesh=mesh,
        out_type=jax.ShapeDtypeStruct((_BATCH, 128), jnp.float32),
        scratch_types=[
            pltpu.VMEM((_BPW,), jnp.int32),
            pltpu.VMEM((_BPW, 128), jnp.float32),
            pltpu.SemaphoreType.DMA,
        ],
    )
    def k(tab_hbm, idx_hbm, out_hbm, idx_v, rows_v, sem):
        wid = lax.axis_index("s") * _NC + lax.axis_index("c")
        base = wid * _BPW
        pltpu.sync_copy(idx_hbm.at[pl.ds(base, _BPW)], idx_v)
        pltpu.async_copy(tab_hbm.at[idx_v], rows_v, sem).wait()
        pltpu.sync_copy(rows_v, out_hbm.at[pl.ds(base, _BPW)])

    return k(packed, ids)


def _project_body(wT_ref, emb_ref, hsel_ref, b_ref, out_ref):
    e = emb_ref[...]
    emb = jnp.where(hsel_ref[...] > 0.5, e[:, _HIDDEN:], e[:, :_HIDDEN])
    acc = lax.dot_general(
        wT_ref[...].astype(jnp.bfloat16),
        emb.astype(jnp.bfloat16),
        (((0,), (1,)), ((), ())),
        preferred_element_type=jnp.float32,
    )
    out_ref[...] = acc + jnp.transpose(b_ref[...])


def _project(wT, emb, hsel, bias2d, interpret=False):
    return pl.pallas_call(
        _project_body,
        grid=(pl.cdiv(_VOCAB, _VT),),
        in_specs=[
            pl.BlockSpec((_HIDDEN, _VT), lambda i: (0, i)),
            pl.BlockSpec((_BATCH, 128), lambda i: (0, 0)),
            pl.BlockSpec((_BATCH, 1), lambda i: (0, 0)),
            pl.BlockSpec((1, _VT), lambda i: (0, i)),
        ],
        out_specs=pl.BlockSpec((_VT, _BATCH), lambda i: (i, 0)),
        out_shape=jax.ShapeDtypeStruct((_VOCAB, _BATCH), jnp.float32),
        interpret=interpret,
    )(wT, emb, hsel, bias2d)


def kernel(input_ids, embedding_weight, linear_weight, linear_bias):
    ids = input_ids.astype(jnp.int32)
    packed = _pack(embedding_weight.T)
    rid = jnp.where(ids < _SPLIT, ids, ids - _SPLIT)
    hsel = (ids >= _SPLIT).astype(jnp.float32).reshape(_BATCH, 1)
    emb128 = _gather_sc(packed, rid)
    bias2d = linear_bias.reshape(1, _VOCAB)
    logitsT = _project(linear_weight.T, emb128, hsel, bias2d)
    return logitsT.T


# speedup vs baseline: 1.1439x; 1.1025x over previous
import functools

import jax
import jax.numpy as jnp
from jax import lax
from jax.experimental import pallas as pl
from jax.experimental.pallas import tpu as pltpu
from jax.experimental.pallas import tpu_sc as plsc

_BATCH = 1024
_HIDDEN = 64
_VOCAB = 100000
_VT = 4096
_PACK_BLK = 3584
_SPLIT = 50176

_NC = 2
_NS = 16
_NW = _NC * _NS
_BPW = _BATCH // _NW


def _pack_body(l_ref, r_ref, out_ref):
    eye = jnp.eye(_HIDDEN, dtype=jnp.bfloat16)
    lT = lax.dot_general(
        l_ref[...].astype(jnp.bfloat16), eye,
        (((0,), (0,)), ((), ())), preferred_element_type=jnp.float32,
    )
    rT = lax.dot_general(
        r_ref[...].astype(jnp.bfloat16), eye,
        (((0,), (0,)), ((), ())), preferred_element_type=jnp.float32,
    )
    out_ref[...] = jnp.concatenate([lT, rT], axis=1)


def _pack(tabT, interpret=False):
    nblk = _SPLIT // _PACK_BLK
    return pl.pallas_call(
        _pack_body,
        grid=(nblk,),
        in_specs=[
            pl.BlockSpec((_HIDDEN, _PACK_BLK), lambda i: (0, i)),
            pl.BlockSpec((_HIDDEN, _PACK_BLK), lambda i: (0, i + nblk)),
        ],
        out_specs=pl.BlockSpec((_PACK_BLK, 128), lambda i: (i, 0)),
        out_shape=jax.ShapeDtypeStruct((_SPLIT, 128), jnp.float32),
        interpret=interpret,
    )(tabT, tabT)


def _gather_sc(packed, ids):
    mesh = plsc.VectorSubcoreMesh(core_axis_name="c", subcore_axis_name="s")

    @functools.partial(
        pl.kernel,
        mesh=mesh,
        out_type=jax.ShapeDtypeStruct((_BATCH, 128), jnp.float32),
        scratch_types=[
            pltpu.VMEM((_BPW,), jnp.int32),
            pltpu.VMEM((_BPW, 128), jnp.float32),
            pltpu.SemaphoreType.DMA,
        ],
    )
    def k(tab_hbm, idx_hbm, out_hbm, idx_v, rows_v, sem):
        wid = lax.axis_index("s") * _NC + lax.axis_index("c")
        base = wid * _BPW
        pltpu.sync_copy(idx_hbm.at[pl.ds(base, _BPW)], idx_v)
        pltpu.async_copy(tab_hbm.at[idx_v], rows_v, sem).wait()
        pltpu.sync_copy(rows_v, out_hbm.at[pl.ds(base, _BPW)])

    return k(packed, ids)


def _project_body(wT_ref, emb_ref, hsel_ref, b_ref, out_ref):
    e = emb_ref[...]
    emb = jnp.where(hsel_ref[...] > 0.5, e[:, _HIDDEN:], e[:, :_HIDDEN])
    acc = lax.dot_general(
        wT_ref[...].astype(jnp.bfloat16),
        emb.astype(jnp.bfloat16),
        (((0,), (1,)), ((), ())),
        preferred_element_type=jnp.float32,
    )
    out_ref[...] = acc + jnp.transpose(b_ref[...])


def _project(wT, emb, hsel, bias2d, interpret=False):
    return pl.pallas_call(
        _project_body,
        grid=(pl.cdiv(_VOCAB, _VT),),
        in_specs=[
            pl.BlockSpec((_HIDDEN, _VT), lambda i: (0, i)),
            pl.BlockSpec((_BATCH, 128), lambda i: (0, 0)),
            pl.BlockSpec((_BATCH, 1), lambda i: (0, 0)),
            pl.BlockSpec((1, _VT), lambda i: (0, i)),
        ],
        out_specs=pl.BlockSpec((_VT, _BATCH), lambda i: (i, 0)),
        out_shape=jax.ShapeDtypeStruct((_VOCAB, _BATCH), jnp.float32),
        interpret=interpret,
    )(wT, emb, hsel, bias2d)


def kernel(input_ids, embedding_weight, linear_weight, linear_bias):
    ids = input_ids.astype(jnp.int32)
    packed = _pack(embedding_weight.T)
    rid = jnp.where(ids < _SPLIT, ids, ids - _SPLIT)
    hsel = (ids >= _SPLIT).astype(jnp.float32).reshape(_BATCH, 1)
    emb128 = _gather_sc(packed, rid)
    bias2d = linear_bias.reshape(1, _VOCAB)
    logitsT = _project(linear_weight.T, emb128, hsel, bias2d)
    return logitsT.T


# speedup vs baseline: 1.1651x; 1.0185x over previous
import functools

import jax
import jax.numpy as jnp
from jax import lax
from jax.experimental import pallas as pl
from jax.experimental.pallas import tpu as pltpu
from jax.experimental.pallas import tpu_sc as plsc

_BATCH = 1024
_HIDDEN = 64
_VOCAB = 100000
_VT = 4096
_PACK_BLK = 7168
_SPLIT = 50176

_NC = 2
_NS = 16
_NW = _NC * _NS
_BPW = _BATCH // _NW


def _pack_body(l_ref, r_ref, out_ref):
    eye = jnp.eye(_HIDDEN, dtype=jnp.bfloat16)
    lT = lax.dot_general(
        l_ref[...].astype(jnp.bfloat16), eye,
        (((0,), (0,)), ((), ())), preferred_element_type=jnp.float32,
    )
    rT = lax.dot_general(
        r_ref[...].astype(jnp.bfloat16), eye,
        (((0,), (0,)), ((), ())), preferred_element_type=jnp.float32,
    )
    out_ref[...] = jnp.concatenate([lT, rT], axis=1)


def _pack(tabT, interpret=False):
    nblk = _SPLIT // _PACK_BLK
    return pl.pallas_call(
        _pack_body,
        grid=(nblk,),
        in_specs=[
            pl.BlockSpec((_HIDDEN, _PACK_BLK), lambda i: (0, i)),
            pl.BlockSpec((_HIDDEN, _PACK_BLK), lambda i: (0, i + nblk)),
        ],
        out_specs=pl.BlockSpec((_PACK_BLK, 128), lambda i: (i, 0)),
        out_shape=jax.ShapeDtypeStruct((_SPLIT, 128), jnp.float32),
        interpret=interpret,
    )(tabT, tabT)


def _gather_sc(packed, ids):
    mesh = plsc.VectorSubcoreMesh(core_axis_name="c", subcore_axis_name="s")

    @functools.partial(
        pl.kernel,
        mesh=mesh,
        out_type=jax.ShapeDtypeStruct((_BATCH, 128), jnp.float32),
        scratch_types=[
            pltpu.VMEM((_BPW,), jnp.int32),
            pltpu.VMEM((_BPW, 128), jnp.float32),
            pltpu.SemaphoreType.DMA,
        ],
    )
    def k(tab_hbm, idx_hbm, out_hbm, idx_v, rows_v, sem):
        wid = lax.axis_index("s") * _NC + lax.axis_index("c")
        base = wid * _BPW
        pltpu.sync_copy(idx_hbm.at[pl.ds(base, _BPW)], idx_v)
        pltpu.async_copy(tab_hbm.at[idx_v], rows_v, sem).wait()
        pltpu.sync_copy(rows_v, out_hbm.at[pl.ds(base, _BPW)])

    return k(packed, ids)


def _project_body(wT_ref, emb_ref, hsel_ref, b_ref, out_ref):
    e = emb_ref[...]
    emb = jnp.where(hsel_ref[...] > 0.5, e[:, _HIDDEN:], e[:, :_HIDDEN])
    acc = lax.dot_general(
        wT_ref[...].astype(jnp.bfloat16),
        emb.astype(jnp.bfloat16),
        (((0,), (1,)), ((), ())),
        preferred_element_type=jnp.float32,
    )
    out_ref[...] = acc + jnp.transpose(b_ref[...])


def _project(wT, emb, hsel, bias2d, interpret=False):
    return pl.pallas_call(
        _project_body,
        grid=(pl.cdiv(_VOCAB, _VT),),
        in_specs=[
            pl.BlockSpec((_HIDDEN, _VT), lambda i: (0, i)),
            pl.BlockSpec((_BATCH, 128), lambda i: (0, 0)),
            pl.BlockSpec((_BATCH, 1), lambda i: (0, 0)),
            pl.BlockSpec((1, _VT), lambda i: (0, i)),
        ],
        out_specs=pl.BlockSpec((_VT, _BATCH), lambda i: (i, 0)),
        out_shape=jax.ShapeDtypeStruct((_VOCAB, _BATCH), jnp.float32),
        interpret=interpret,
    )(wT, emb, hsel, bias2d)


def kernel(input_ids, embedding_weight, linear_weight, linear_bias):
    ids = input_ids.astype(jnp.int32)
    packed = _pack(embedding_weight.T)
    rid = jnp.where(ids < _SPLIT, ids, ids - _SPLIT)
    hsel = (ids >= _SPLIT).astype(jnp.float32).reshape(_BATCH, 1)
    emb128 = _gather_sc(packed, rid)
    bias2d = linear_bias.reshape(1, _VOCAB)
    logitsT = _project(linear_weight.T, emb128, hsel, bias2d)
    return logitsT.T


# speedup vs baseline: 1.1665x; 1.0013x over previous
import functools

import jax
import jax.numpy as jnp
from jax import lax
from jax.experimental import pallas as pl
from jax.experimental.pallas import tpu as pltpu
from jax.experimental.pallas import tpu_sc as plsc

_BATCH = 1024
_HIDDEN = 64
_VOCAB = 100000
_VT = 4096
_PACK_BLK = 12544
_SPLIT = 50176

_NC = 2
_NS = 16
_NW = _NC * _NS
_BPW = _BATCH // _NW


def _pack_body(l_ref, r_ref, out_ref):
    eye = jnp.eye(_HIDDEN, dtype=jnp.bfloat16)
    lT = lax.dot_general(
        l_ref[...].astype(jnp.bfloat16), eye,
        (((0,), (0,)), ((), ())), preferred_element_type=jnp.float32,
    )
    rT = lax.dot_general(
        r_ref[...].astype(jnp.bfloat16), eye,
        (((0,), (0,)), ((), ())), preferred_element_type=jnp.float32,
    )
    out_ref[...] = jnp.concatenate([lT, rT], axis=1)


def _pack(tabT, interpret=False):
    nblk = _SPLIT // _PACK_BLK
    return pl.pallas_call(
        _pack_body,
        grid=(nblk,),
        in_specs=[
            pl.BlockSpec((_HIDDEN, _PACK_BLK), lambda i: (0, i)),
            pl.BlockSpec((_HIDDEN, _PACK_BLK), lambda i: (0, i + nblk)),
        ],
        out_specs=pl.BlockSpec((_PACK_BLK, 128), lambda i: (i, 0)),
        out_shape=jax.ShapeDtypeStruct((_SPLIT, 128), jnp.float32),
        interpret=interpret,
    )(tabT, tabT)


def _gather_sc(packed, ids):
    mesh = plsc.VectorSubcoreMesh(core_axis_name="c", subcore_axis_name="s")

    @functools.partial(
        pl.kernel,
        mesh=mesh,
        out_type=jax.ShapeDtypeStruct((_BATCH, 128), jnp.float32),
        scratch_types=[
            pltpu.VMEM((_BPW,), jnp.int32),
            pltpu.VMEM((_BPW, 128), jnp.float32),
            pltpu.SemaphoreType.DMA,
        ],
    )
    def k(tab_hbm, idx_hbm, out_hbm, idx_v, rows_v, sem):
        wid = lax.axis_index("s") * _NC + lax.axis_index("c")
        base = wid * _BPW
        pltpu.sync_copy(idx_hbm.at[pl.ds(base, _BPW)], idx_v)
        pltpu.async_copy(tab_hbm.at[idx_v], rows_v, sem).wait()
        pltpu.sync_copy(rows_v, out_hbm.at[pl.ds(base, _BPW)])

    return k(packed, ids)


def _project_body(wT_ref, emb_ref, hsel_ref, b_ref, out_ref):
    e = emb_ref[...]
    emb = jnp.where(hsel_ref[...] > 0.5, e[:, _HIDDEN:], e[:, :_HIDDEN])
    acc = lax.dot_general(
        wT_ref[...].astype(jnp.bfloat16),
        emb.astype(jnp.bfloat16),
        (((0,), (1,)), ((), ())),
        preferred_element_type=jnp.float32,
    )
    out_ref[...] = acc + jnp.transpose(b_ref[...])


def _project(wT, emb, hsel, bias2d, interpret=False):
    return pl.pallas_call(
        _project_body,
        grid=(pl.cdiv(_VOCAB, _VT),),
        in_specs=[
            pl.BlockSpec((_HIDDEN, _VT), lambda i: (0, i)),
            pl.BlockSpec((_BATCH, 128), lambda i: (0, 0)),
            pl.BlockSpec((_BATCH, 1), lambda i: (0, 0)),
            pl.BlockSpec((1, _VT), lambda i: (0, i)),
        ],
        out_specs=pl.BlockSpec((_VT, _BATCH), lambda i: (i, 0)),
        out_shape=jax.ShapeDtypeStruct((_VOCAB, _BATCH), jnp.float32),
        interpret=interpret,
    )(wT, emb, hsel, bias2d)


def kernel(input_ids, embedding_weight, linear_weight, linear_bias):
    ids = input_ids.astype(jnp.int32)
    packed = _pack(embedding_weight.T)
    rid = jnp.where(ids < _SPLIT, ids, ids - _SPLIT)
    hsel = (ids >= _SPLIT).astype(jnp.float32).reshape(_BATCH, 1)
    emb128 = _gather_sc(packed, rid)
    bias2d = linear_bias.reshape(1, _VOCAB)
    logitsT = _project(linear_weight.T, emb128, hsel, bias2d)
    return logitsT.T


# speedup vs baseline: 1.1708x; 1.0036x over previous
import functools

import jax
import jax.numpy as jnp
from jax import lax
from jax.experimental import pallas as pl
from jax.experimental.pallas import tpu as pltpu
from jax.experimental.pallas import tpu_sc as plsc

_BATCH = 1024
_HIDDEN = 64
_VOCAB = 100000
_VT = 5120
_PACK_BLK = 12544
_SPLIT = 50176

_NC = 2
_NS = 16
_NW = _NC * _NS
_BPW = _BATCH // _NW


def _pack_body(l_ref, r_ref, out_ref):
    eye = jnp.eye(_HIDDEN, dtype=jnp.bfloat16)
    lT = lax.dot_general(
        l_ref[...].astype(jnp.bfloat16), eye,
        (((0,), (0,)), ((), ())), preferred_element_type=jnp.float32,
    )
    rT = lax.dot_general(
        r_ref[...].astype(jnp.bfloat16), eye,
        (((0,), (0,)), ((), ())), preferred_element_type=jnp.float32,
    )
    out_ref[...] = jnp.concatenate([lT, rT], axis=1)


def _pack(tabT, interpret=False):
    nblk = _SPLIT // _PACK_BLK
    return pl.pallas_call(
        _pack_body,
        grid=(nblk,),
        in_specs=[
            pl.BlockSpec((_HIDDEN, _PACK_BLK), lambda i: (0, i)),
            pl.BlockSpec((_HIDDEN, _PACK_BLK), lambda i: (0, i + nblk)),
        ],
        out_specs=pl.BlockSpec((_PACK_BLK, 128), lambda i: (i, 0)),
        out_shape=jax.ShapeDtypeStruct((_SPLIT, 128), jnp.float32),
        interpret=interpret,
    )(tabT, tabT)


def _gather_sc(packed, ids):
    mesh = plsc.VectorSubcoreMesh(core_axis_name="c", subcore_axis_name="s")

    @functools.partial(
        pl.kernel,
        mesh=mesh,
        out_type=jax.ShapeDtypeStruct((_BATCH, 128), jnp.float32),
        scratch_types=[
            pltpu.VMEM((_BPW,), jnp.int32),
            pltpu.VMEM((_BPW, 128), jnp.float32),
            pltpu.SemaphoreType.DMA,
        ],
    )
    def k(tab_hbm, idx_hbm, out_hbm, idx_v, rows_v, sem):
        wid = lax.axis_index("s") * _NC + lax.axis_index("c")
        base = wid * _BPW
        pltpu.sync_copy(idx_hbm.at[pl.ds(base, _BPW)], idx_v)
        pltpu.async_copy(tab_hbm.at[idx_v], rows_v, sem).wait()
        pltpu.sync_copy(rows_v, out_hbm.at[pl.ds(base, _BPW)])

    return k(packed, ids)


def _project_body(wT_ref, emb_ref, hsel_ref, b_ref, out_ref):
    e = emb_ref[...]
    emb = jnp.where(hsel_ref[...] > 0.5, e[:, _HIDDEN:], e[:, :_HIDDEN])
    acc = lax.dot_general(
        wT_ref[...].astype(jnp.bfloat16),
        emb.astype(jnp.bfloat16),
        (((0,), (1,)), ((), ())),
        preferred_element_type=jnp.float32,
    )
    out_ref[...] = acc + jnp.transpose(b_ref[...])


def _project(wT, emb, hsel, bias2d, interpret=False):
    return pl.pallas_call(
        _project_body,
        grid=(pl.cdiv(_VOCAB, _VT),),
        in_specs=[
            pl.BlockSpec((_HIDDEN, _VT), lambda i: (0, i)),
            pl.BlockSpec((_BATCH, 128), lambda i: (0, 0)),
            pl.BlockSpec((_BATCH, 1), lambda i: (0, 0)),
            pl.BlockSpec((1, _VT), lambda i: (0, i)),
        ],
        out_specs=pl.BlockSpec((_VT, _BATCH), lambda i: (i, 0)),
        out_shape=jax.ShapeDtypeStruct((_VOCAB, _BATCH), jnp.float32),
        interpret=interpret,
    )(wT, emb, hsel, bias2d)


def kernel(input_ids, embedding_weight, linear_weight, linear_bias):
    ids = input_ids.astype(jnp.int32)
    packed = _pack(embedding_weight.T)
    rid = jnp.where(ids < _SPLIT, ids, ids - _SPLIT)
    hsel = (ids >= _SPLIT).astype(jnp.float32).reshape(_BATCH, 1)
    emb128 = _gather_sc(packed, rid)
    bias2d = linear_bias.reshape(1, _VOCAB)
    logitsT = _project(linear_weight.T, emb128, hsel, bias2d)
    return logitsT.T
